# baseline (speedup 1.0000x reference)
import jax
import jax.numpy as jnp
from jax import lax
from jax.experimental import pallas as pl

B = 4096
T = 200
D = 128
V = 60

BB = 256
GRID = B // BB


def _tc_body(t_ref, table_ref, out_ref):
    idx = t_ref[0, 0, :]
    onehot = (idx[:, None] == lax.broadcasted_iota(jnp.int32, (BB, V), 1)
              ).astype(jnp.float32)
    emb = jnp.dot(onehot, table_ref[...],
                  preferred_element_type=jnp.float32,
                  precision=lax.Precision.HIGHEST)
    out_ref[...] = jnp.broadcast_to(emb[:, None, :], (BB, T, D))


@jax.jit
def _run(t, table):
    t3 = t.reshape(GRID, 1, BB)
    return pl.pallas_call(
        _tc_body,
        grid=(GRID,),
        in_specs=[
            pl.BlockSpec((1, 1, BB), lambda i: (i, 0, 0)),
            pl.BlockSpec((V, D), lambda i: (0, 0)),
        ],
        out_specs=pl.BlockSpec((BB, T, D), lambda i: (i, 0, 0)),
        out_shape=jax.ShapeDtypeStruct((B, T, D), jnp.float32),
    )(t3, table)


def kernel(t, n_tokens, table):
    del n_tokens
    return _run(t, table)


# speedup vs baseline: 1.0095x; 1.0095x over previous
import jax
import jax.numpy as jnp
from jax import lax
from jax.experimental import pallas as pl

B = 4096
T = 200
D = 128
V = 60

BB = 64
GRID = B // BB


def _tc_body(t_ref, table_ref, out_ref):
    idx = t_ref[0, 0, :]
    onehot = (idx[:, None] == lax.broadcasted_iota(jnp.int32, (BB, V), 1)
              ).astype(jnp.float32)
    emb = jnp.dot(onehot, table_ref[...],
                  preferred_element_type=jnp.float32,
                  precision=lax.Precision.HIGHEST)
    out_ref[...] = jnp.broadcast_to(emb[:, None, :], (BB, T, D))


@jax.jit
def _run(t, table):
    t3 = t.reshape(GRID, 1, BB)
    return pl.pallas_call(
        _tc_body,
        grid=(GRID,),
        in_specs=[
            pl.BlockSpec((1, 1, BB), lambda i: (i, 0, 0)),
            pl.BlockSpec((V, D), lambda i: (0, 0)),
        ],
        out_specs=pl.BlockSpec((BB, T, D), lambda i: (i, 0, 0)),
        out_shape=jax.ShapeDtypeStruct((B, T, D), jnp.float32),
    )(t3, table)


def kernel(t, n_tokens, table):
    del n_tokens
    return _run(t, table)
